# baseline (speedup 1.0000x reference)
import functools

import jax
import jax.numpy as jnp
from jax import lax
from jax.experimental import pallas as pl
from jax.experimental.pallas import tpu as pltpu
from jax.experimental.pallas import tpu_sc as plsc

_POOL = 10
_TOPK = 2
_BB = 8


def _eprompt_body(x_ref, pk_ref,
                  sim_ref, idx_ref, pkn_ref, xn_ref, rs_ref):
    xm = jnp.max(x_ref[...], axis=1)
    xss = jnp.sum(xm * xm, axis=-1, keepdims=True)
    xn = xm * jax.lax.rsqrt(jnp.maximum(xss, 1e-12))
    pk = pk_ref[...]
    pss = jnp.sum(pk * pk, axis=-1, keepdims=True)
    pkn = pk * jax.lax.rsqrt(jnp.maximum(pss, 1e-12))
    pkn_ref[...] = pkn
    xn_ref[...] = xn
    sim = jax.lax.dot_general(xn, pkn, (((1,), (1,)), ((), ())),
                              preferred_element_type=jnp.float32)
    sim_ref[...] = sim
    cols = jax.lax.broadcasted_iota(jnp.int32, sim.shape, 1)
    v1 = jnp.max(sim, axis=1, keepdims=True)
    i1 = jnp.min(jnp.where(sim == v1, cols, _POOL), axis=1, keepdims=True)
    sim_m = jnp.where(cols == i1, -jnp.inf, sim)
    v2 = jnp.max(sim_m, axis=1, keepdims=True)
    i2 = jnp.min(jnp.where(sim_m == v2, cols, _POOL), axis=1, keepdims=True)
    idx_ref[...] = jnp.concatenate([i1, i2], axis=1)

    @pl.when(pl.program_id(0) == 0)
    def _():
        rs_ref[...] = jnp.zeros_like(rs_ref)

    rs_ref[...] = rs_ref[...] + (jnp.sum(v1) + jnp.sum(v2))


def _tc_stage(x_embed, prompt_key):
    B, L, D = x_embed.shape
    return pl.pallas_call(
        _eprompt_body,
        grid=(B // _BB,),
        in_specs=[
            pl.BlockSpec((_BB, L, D), lambda i: (i, 0, 0)),
            pl.BlockSpec((_POOL, D), lambda i: (0, 0)),
        ],
        out_specs=[
            pl.BlockSpec((_BB, _POOL), lambda i: (i, 0)),
            pl.BlockSpec((_BB, _TOPK), lambda i: (i, 0)),
            pl.BlockSpec((_POOL, D), lambda i: (0, 0)),
            pl.BlockSpec((_BB, D), lambda i: (i, 0)),
            pl.BlockSpec((1, 1), lambda i: (0, 0)),
        ],
        out_shape=[
            jax.ShapeDtypeStruct((B, _POOL), jnp.float32),
            jax.ShapeDtypeStruct((B, _TOPK), jnp.int32),
            jax.ShapeDtypeStruct((_POOL, D), jnp.float32),
            jax.ShapeDtypeStruct((B, D), jnp.float32),
            jax.ShapeDtypeStruct((1, 1), jnp.float32),
        ],
    )(x_embed, prompt_key)


def _make_sc_gather(n_idx, D):
    info = plsc.get_sparse_core_info()
    nw = info.num_cores * info.num_subcores
    bpw = n_idx // nw
    mesh = plsc.VectorSubcoreMesh(core_axis_name="c", subcore_axis_name="s")

    @functools.partial(
        pl.kernel, mesh=mesh,
        out_type=[jax.ShapeDtypeStruct((n_idx, D), jnp.float32),
                  jax.ShapeDtypeStruct((n_idx, D), jnp.float32)],
        scratch_types=[
            pltpu.VMEM((bpw,), jnp.int32),
            pltpu.VMEM((bpw, D), jnp.float32),
            pltpu.VMEM((bpw, D), jnp.float32),
            pltpu.SemaphoreType.DMA,
            pltpu.SemaphoreType.DMA,
        ],
    )
    def sc_gather(pkn_hbm, p_hbm, idx_hbm, bkn_hbm, bp_hbm,
                  idx_v, rows_k, rows_p, sem_k, sem_p):
        wid = lax.axis_index("s") * info.num_cores + lax.axis_index("c")
        base = wid * bpw
        pltpu.sync_copy(idx_hbm.at[pl.ds(base, bpw)], idx_v)
        ck = pltpu.async_copy(pkn_hbm.at[idx_v], rows_k, sem_k)
        cp = pltpu.async_copy(p_hbm.at[idx_v], rows_p, sem_p)
        ck.wait()
        cp.wait()
        pltpu.sync_copy(rows_k, bkn_hbm.at[pl.ds(base, bpw)])
        pltpu.sync_copy(rows_p, bp_hbm.at[pl.ds(base, bpw)])

    return sc_gather


def kernel(x_embed, prompt, prompt_key):
    B, L, D = x_embed.shape
    sim, idx, pkn, xn, rs = _tc_stage(x_embed, prompt_key)
    idx_flat = idx.reshape(B * _TOPK)
    bkn_f, bp_f = _make_sc_gather(B * _TOPK, D)(pkn, prompt, idx_flat)
    bkn = bkn_f.reshape(B, _TOPK, D)
    bp = bp_f.reshape(B, _TOPK, D)
    reduce_sim = (rs[0, 0] / B).astype(jnp.float32).reshape(())
    return (sim, idx, bkn, pkn, xn, reduce_sim, bp)
